# baseline (speedup 1.0000x reference)
import functools

import jax
import jax.numpy as jnp
from jax import lax
from jax.experimental import pallas as pl
from jax.experimental.pallas import tpu as pltpu
from jax.experimental.pallas import tpu_sc as plsc

N = 1024
B = 8
GRU = 64
PATCH = 12
NPATCH = 12
OUTW = 12
IB = GRU // 2


def _dot(a, b):
    return jax.lax.dot_general(a.astype(jnp.bfloat16), b.astype(jnp.bfloat16),
                               (((1,), (0,)), ((), ())),
                               preferred_element_type=jnp.float32)


def _step_body(psrc_ref, L_ref, bdve_ref, bd0r_i_ref, bd0r_s_ref,
               bd0u_i_ref, bd0u_s_ref, bd1_i_ref, bd1_s_ref, bdout_ref,
               pe_ref, b0r_ref, b0u_ref, b1_ref, bout_ref,
               out_ref, S_ref):
    t = pl.program_id(0)

    @pl.when(t == 0)
    def _():
        S_ref[...] = jnp.zeros_like(S_ref)

    L = L_ref[...]
    P = psrc_ref[0]
    Xe = _dot(P, bdve_ref[...]) + pe_ref[0]
    LXe = _dot(L, Xe)
    S = S_ref[...]
    SA = _dot(L, S)
    r = jax.nn.sigmoid(_dot(LXe, bd0r_i_ref[...]) + _dot(SA, bd0r_s_ref[...])
                       + b0r_ref[...])
    u = jax.nn.sigmoid(_dot(LXe, bd0u_i_ref[...]) + _dot(SA, bd0u_s_ref[...])
                       + b0u_ref[...])
    SB = _dot(L, r * S)
    c = jnp.tanh(_dot(LXe, bd1_i_ref[...]) + _dot(SB, bd1_s_ref[...])
                 + b1_ref[...])
    Snew = u * S + (1.0 - u) * c
    S_ref[...] = Snew

    @pl.when(t == NPATCH - 1)
    def _():
        out_ref[...] = _dot(Snew, bdout_ref[...]) + bout_ref[...]


def _bd(W):
    return jnp.kron(jnp.eye(B, dtype=W.dtype), W)


_NWORK = 32
_LANES = 16
_ROWS_PER_W = N // _NWORK
_SLAB = _ROWS_PER_W * N


def _densify_coo(adj_row, adj_col, adj_val):
    nnz = adj_row.shape[0]
    e_pad = ((nnz + _LANES - 1) // _LANES) * _LANES
    pad = e_pad - nnz
    rows = jnp.concatenate(
        [adj_row.astype(jnp.int32), jnp.zeros((pad,), jnp.int32)])
    cols = jnp.concatenate(
        [adj_col.astype(jnp.int32), jnp.zeros((pad,), jnp.int32)])
    vals = jnp.concatenate([adj_val, jnp.zeros((pad,), jnp.float32)])
    zeros_slab = jnp.zeros((_SLAB,), jnp.float32)

    mesh = plsc.VectorSubcoreMesh(core_axis_name="c", subcore_axis_name="s")

    @functools.partial(
        pl.kernel, mesh=mesh,
        out_type=jax.ShapeDtypeStruct((N * N,), jnp.float32),
        scratch_types=[
            pltpu.VMEM((e_pad,), jnp.int32),
            pltpu.VMEM((e_pad,), jnp.int32),
            pltpu.VMEM((e_pad,), jnp.float32),
            pltpu.VMEM((_SLAB,), jnp.float32),
        ],
        compiler_params=pltpu.CompilerParams(needs_layout_passes=False),
    )
    def _scatter(row_hbm, col_hbm, val_hbm, zero_hbm, L_hbm, rv, cv, vv, slab):
        wid = lax.axis_index("s") * 2 + lax.axis_index("c")
        base = wid * _SLAB
        pltpu.sync_copy(row_hbm, rv)
        pltpu.sync_copy(col_hbm, cv)
        pltpu.sync_copy(val_hbm, vv)
        pltpu.sync_copy(zero_hbm, slab)

        def ebody(i, carry):
            r16 = rv[pl.ds(i * _LANES, _LANES)]
            c16 = cv[pl.ds(i * _LANES, _LANES)]
            v16 = vv[pl.ds(i * _LANES, _LANES)]
            fl = r16 * N + c16 - base
            ok = (fl >= 0) & (fl < _SLAB)
            fl = jnp.where(ok, fl, 0)
            v16 = jnp.where(ok, v16, jnp.float32(0.0))
            plsc.addupdate_scatter(slab, [fl], v16)
            return carry

        lax.fori_loop(0, e_pad // _LANES, ebody, 0)
        pltpu.sync_copy(slab, L_hbm.at[pl.ds(base, _SLAB)])

    return _scatter(rows, cols, vals, zeros_slab).reshape(N, N)


def kernel(source, select_dataset, W_ve, pe, W0, b0, W1, b1, W_out, b_out,
           adj_row, adj_col, adj_val):
    f32 = jnp.float32

    Ld = _densify_coo(adj_row, adj_col, adj_val)

    Psrc = jnp.transpose(
        jnp.squeeze(source, -1).reshape(B, NPATCH, PATCH, N),
        (1, 3, 0, 2)).reshape(NPATCH, N, B * PATCH)

    bdve = _bd(W_ve)
    bd0r_i = _bd(W0[:GRU, :GRU])
    bd0r_s = _bd(W0[GRU:, :GRU])
    bd0u_i = _bd(W0[:GRU, GRU:])
    bd0u_s = _bd(W0[GRU:, GRU:])
    bd1_i = _bd(W1[:GRU, :])
    bd1_s = _bd(W1[GRU:, :])
    W_out_pad = jnp.concatenate(
        [W_out, jnp.zeros((GRU - IB, OUTW), f32)], axis=0)
    bdout = _bd(W_out_pad)

    petile = jnp.tile(pe[0, :NPATCH], (1, B)).reshape(NPATCH, 1, B * GRU)
    b0r = jnp.tile(b0[:GRU], B)[None]
    b0u = jnp.tile(b0[GRU:], B)[None]
    b1t = jnp.tile(b1, B)[None]
    boutt = jnp.tile(b_out, B)[None]

    full = lambda shape: pl.BlockSpec(shape, lambda t: tuple(0 for _ in shape))
    out = pl.pallas_call(
        _step_body,
        grid=(NPATCH,),
        in_specs=[
            pl.BlockSpec((1, N, B * PATCH), lambda t: (t, 0, 0)),
            full((N, N)),
            full((B * PATCH, B * GRU)),
            full((B * GRU, B * GRU)),
            full((B * GRU, B * GRU)),
            full((B * GRU, B * GRU)),
            full((B * GRU, B * GRU)),
            full((B * GRU, B * GRU)),
            full((B * GRU, B * GRU)),
            full((B * GRU, B * OUTW)),
            pl.BlockSpec((1, 1, B * GRU), lambda t: (t, 0, 0)),
            full((1, B * GRU)),
            full((1, B * GRU)),
            full((1, B * GRU)),
            full((1, B * OUTW)),
        ],
        out_specs=pl.BlockSpec((N, B * OUTW), lambda t: (0, 0)),
        out_shape=jax.ShapeDtypeStruct((N, B * OUTW), f32),
        scratch_shapes=[pltpu.VMEM((N, B * GRU), f32)],
        compiler_params=pltpu.CompilerParams(
            dimension_semantics=("arbitrary",)),
    )(Psrc, Ld, bdve, bd0r_i, bd0r_s, bd0u_i, bd0u_s, bd1_i, bd1_s, bdout,
      petile, b0r, b0u, b1t, boutt)

    return jnp.transpose(out.reshape(N, B, OUTW), (1, 2, 0))[..., None]


# speedup vs baseline: 1.0008x; 1.0008x over previous
import functools

import jax
import jax.numpy as jnp
from jax import lax
from jax.experimental import pallas as pl
from jax.experimental.pallas import tpu as pltpu
from jax.experimental.pallas import tpu_sc as plsc

N = 1024
B = 8
GRU = 64
PATCH = 12
NPATCH = 12
OUTW = 12
IB = GRU // 2


def _dot(a, b):
    return jax.lax.dot_general(a.astype(jnp.bfloat16), b.astype(jnp.bfloat16),
                               (((1,), (0,)), ((), ())),
                               preferred_element_type=jnp.float32)


def _step_body(psrc_ref, L_ref, bdve_ref, bd0r_i_ref, bd0r_s_ref,
               bd0u_i_ref, bd0u_s_ref, bd1_i_ref, bd1_s_ref, bdout_ref,
               pe_ref, b0r_ref, b0u_ref, b1_ref, bout_ref,
               out_ref, S_ref):
    t = pl.program_id(0)

    @pl.when(t == 0)
    def _():
        S_ref[...] = jnp.zeros_like(S_ref)

    L = L_ref[...]
    P = psrc_ref[0]
    Xe = _dot(P, bdve_ref[...]) + pe_ref[0]
    LXe = _dot(L, Xe)
    S = S_ref[...]
    SA = _dot(L, S)
    r = jax.nn.sigmoid(_dot(LXe, bd0r_i_ref[...]) + _dot(SA, bd0r_s_ref[...])
                       + b0r_ref[...])
    u = jax.nn.sigmoid(_dot(LXe, bd0u_i_ref[...]) + _dot(SA, bd0u_s_ref[...])
                       + b0u_ref[...])
    SB = _dot(L, r * S)
    c = jnp.tanh(_dot(LXe, bd1_i_ref[...]) + _dot(SB, bd1_s_ref[...])
                 + b1_ref[...])
    Snew = u * S + (1.0 - u) * c
    S_ref[...] = Snew

    @pl.when(t == NPATCH - 1)
    def _():
        out_ref[...] = _dot(Snew, bdout_ref[...]) + bout_ref[...]


def _bd(W):
    return jnp.kron(jnp.eye(B, dtype=W.dtype), W)


_NWORK = 32
_LANES = 16
_ROWS_PER_W = N // _NWORK
_SLAB = _ROWS_PER_W * N


def _densify_coo(adj_row, adj_col, adj_val):
    nnz = adj_row.shape[0]
    e_pad = ((nnz + _LANES - 1) // _LANES) * _LANES
    pad = e_pad - nnz
    rows = jnp.concatenate(
        [adj_row.astype(jnp.int32), jnp.zeros((pad,), jnp.int32)])
    cols = jnp.concatenate(
        [adj_col.astype(jnp.int32), jnp.zeros((pad,), jnp.int32)])
    vals = jnp.concatenate([adj_val, jnp.zeros((pad,), jnp.float32)])
    zeros_slab = jnp.zeros((_SLAB,), jnp.float32)

    mesh = plsc.VectorSubcoreMesh(core_axis_name="c", subcore_axis_name="s")

    @functools.partial(
        pl.kernel, mesh=mesh,
        out_type=jax.ShapeDtypeStruct((N * N,), jnp.float32),
        scratch_types=[
            pltpu.VMEM((e_pad,), jnp.int32),
            pltpu.VMEM((e_pad,), jnp.int32),
            pltpu.VMEM((e_pad,), jnp.float32),
            pltpu.VMEM((_SLAB,), jnp.float32),
        ],
        compiler_params=pltpu.CompilerParams(needs_layout_passes=False),
    )
    def _scatter(row_hbm, col_hbm, val_hbm, zero_hbm, L_hbm, rv, cv, vv, slab):
        wid = lax.axis_index("s") * 2 + lax.axis_index("c")
        base = wid * _SLAB
        pltpu.sync_copy(row_hbm, rv)
        pltpu.sync_copy(col_hbm, cv)
        pltpu.sync_copy(val_hbm, vv)
        pltpu.sync_copy(zero_hbm, slab)

        def ebody(i, carry):
            r16 = rv[pl.ds(i * _LANES, _LANES)]
            c16 = cv[pl.ds(i * _LANES, _LANES)]
            v16 = vv[pl.ds(i * _LANES, _LANES)]
            fl = r16 * N + c16 - base
            ok = (fl >= 0) & (fl < _SLAB)
            fl = jnp.where(ok, fl, 0)
            plsc.addupdate_scatter(slab, [fl], v16, mask=ok)
            return carry

        lax.fori_loop(0, e_pad // _LANES, ebody, 0)
        pltpu.sync_copy(slab, L_hbm.at[pl.ds(base, _SLAB)])

    return _scatter(rows, cols, vals, zeros_slab).reshape(N, N)


def kernel(source, select_dataset, W_ve, pe, W0, b0, W1, b1, W_out, b_out,
           adj_row, adj_col, adj_val):
    f32 = jnp.float32

    Ld = _densify_coo(adj_row, adj_col, adj_val)

    Psrc = jnp.transpose(
        jnp.squeeze(source, -1).reshape(B, NPATCH, PATCH, N),
        (1, 3, 0, 2)).reshape(NPATCH, N, B * PATCH)

    bdve = _bd(W_ve)
    bd0r_i = _bd(W0[:GRU, :GRU])
    bd0r_s = _bd(W0[GRU:, :GRU])
    bd0u_i = _bd(W0[:GRU, GRU:])
    bd0u_s = _bd(W0[GRU:, GRU:])
    bd1_i = _bd(W1[:GRU, :])
    bd1_s = _bd(W1[GRU:, :])
    W_out_pad = jnp.concatenate(
        [W_out, jnp.zeros((GRU - IB, OUTW), f32)], axis=0)
    bdout = _bd(W_out_pad)

    petile = jnp.tile(pe[0, :NPATCH], (1, B)).reshape(NPATCH, 1, B * GRU)
    b0r = jnp.tile(b0[:GRU], B)[None]
    b0u = jnp.tile(b0[GRU:], B)[None]
    b1t = jnp.tile(b1, B)[None]
    boutt = jnp.tile(b_out, B)[None]

    full = lambda shape: pl.BlockSpec(shape, lambda t: tuple(0 for _ in shape))
    out = pl.pallas_call(
        _step_body,
        grid=(NPATCH,),
        in_specs=[
            pl.BlockSpec((1, N, B * PATCH), lambda t: (t, 0, 0)),
            full((N, N)),
            full((B * PATCH, B * GRU)),
            full((B * GRU, B * GRU)),
            full((B * GRU, B * GRU)),
            full((B * GRU, B * GRU)),
            full((B * GRU, B * GRU)),
            full((B * GRU, B * GRU)),
            full((B * GRU, B * GRU)),
            full((B * GRU, B * OUTW)),
            pl.BlockSpec((1, 1, B * GRU), lambda t: (t, 0, 0)),
            full((1, B * GRU)),
            full((1, B * GRU)),
            full((1, B * GRU)),
            full((1, B * OUTW)),
        ],
        out_specs=pl.BlockSpec((N, B * OUTW), lambda t: (0, 0)),
        out_shape=jax.ShapeDtypeStruct((N, B * OUTW), f32),
        scratch_shapes=[pltpu.VMEM((N, B * GRU), f32)],
        compiler_params=pltpu.CompilerParams(
            dimension_semantics=("arbitrary",)),
    )(Psrc, Ld, bdve, bd0r_i, bd0r_s, bd0u_i, bd0u_s, bd1_i, bd1_s, bdout,
      petile, b0r, b0u, b1t, boutt)

    return jnp.transpose(out.reshape(N, B, OUTW), (1, 2, 0))[..., None]


# speedup vs baseline: 1.1824x; 1.1815x over previous
import functools

import jax
import jax.numpy as jnp
from jax import lax
from jax.experimental import pallas as pl
from jax.experimental.pallas import tpu as pltpu
from jax.experimental.pallas import tpu_sc as plsc

N = 1024
B = 8
GRU = 64
PATCH = 12
NPATCH = 12
OUTW = 12
IB = GRU // 2


def _dot(a, b):
    return jax.lax.dot_general(a.astype(jnp.bfloat16), b.astype(jnp.bfloat16),
                               (((1,), (0,)), ((), ())),
                               preferred_element_type=jnp.float32)


def _step_body(psrc_ref, L_ref, bdve_ref, bd0r_i_ref, bd0r_s_ref,
               bd0u_i_ref, bd0u_s_ref, bd1_i_ref, bd1_s_ref, bdout_ref,
               pe_ref, out_ref, S_ref):
    t = pl.program_id(0)

    @pl.when(t == 0)
    def _():
        S_ref[...] = jnp.zeros_like(S_ref)

    L = L_ref[...]
    P = psrc_ref[0]
    Xe = _dot(P, bdve_ref[...]) + pe_ref[0]
    LXe = _dot(L, Xe)
    S = S_ref[...]
    SA = _dot(L, S)
    r = jax.nn.sigmoid(_dot(LXe, bd0r_i_ref[...]) + _dot(SA, bd0r_s_ref[...]))
    u = jax.nn.sigmoid(_dot(LXe, bd0u_i_ref[...]) + _dot(SA, bd0u_s_ref[...]))
    SB = _dot(L, r * S)
    c = jnp.tanh(_dot(LXe, bd1_i_ref[...]) + _dot(SB, bd1_s_ref[...]))
    Snew = u * S + (1.0 - u) * c
    S_ref[...] = Snew

    @pl.when(t == NPATCH - 1)
    def _():
        out_ref[...] = _dot(Snew, bdout_ref[...])


def _bd(W):
    return jnp.kron(jnp.eye(B, dtype=W.dtype), W)


_NWORK = 32
_LANES = 16
_ROWS_PER_W = N // _NWORK
_SLAB = _ROWS_PER_W * N


def _densify_coo(adj_row, adj_col, adj_val):
    nnz = adj_row.shape[0]
    e_pad = ((nnz + _LANES - 1) // _LANES) * _LANES
    pad = e_pad - nnz
    rows = jnp.concatenate(
        [adj_row.astype(jnp.int32), jnp.zeros((pad,), jnp.int32)])
    cols = jnp.concatenate(
        [adj_col.astype(jnp.int32), jnp.zeros((pad,), jnp.int32)])
    vals = jnp.concatenate([adj_val, jnp.zeros((pad,), jnp.float32)])
    zeros_slab = jnp.zeros((_SLAB,), jnp.float32)

    mesh = plsc.VectorSubcoreMesh(core_axis_name="c", subcore_axis_name="s")

    @functools.partial(
        pl.kernel, mesh=mesh,
        out_type=jax.ShapeDtypeStruct((N * N,), jnp.float32),
        scratch_types=[
            pltpu.VMEM((e_pad,), jnp.int32),
            pltpu.VMEM((e_pad,), jnp.int32),
            pltpu.VMEM((e_pad,), jnp.float32),
            pltpu.VMEM((_SLAB,), jnp.float32),
        ],
        compiler_params=pltpu.CompilerParams(needs_layout_passes=False),
    )
    def _scatter(row_hbm, col_hbm, val_hbm, zero_hbm, L_hbm, rv, cv, vv, slab):
        wid = lax.axis_index("s") * 2 + lax.axis_index("c")
        base = wid * _SLAB
        pltpu.sync_copy(row_hbm, rv)
        pltpu.sync_copy(col_hbm, cv)
        pltpu.sync_copy(val_hbm, vv)
        pltpu.sync_copy(zero_hbm, slab)

        def ebody(i, carry):
            r16 = rv[pl.ds(i * _LANES, _LANES)]
            c16 = cv[pl.ds(i * _LANES, _LANES)]
            v16 = vv[pl.ds(i * _LANES, _LANES)]
            fl = r16 * N + c16 - base
            ok = (fl >= 0) & (fl < _SLAB)
            fl = jnp.where(ok, fl, 0)
            plsc.addupdate_scatter(slab, [fl], v16, mask=ok)
            return carry

        lax.fori_loop(0, e_pad // _LANES, ebody, 0)
        pltpu.sync_copy(slab, L_hbm.at[pl.ds(base, _SLAB)])

    return _scatter(rows, cols, vals, zeros_slab).reshape(N, N)


def kernel(source, select_dataset, W_ve, pe, W0, b0, W1, b1, W_out, b_out,
           adj_row, adj_col, adj_val):
    f32 = jnp.float32

    bf16 = jnp.bfloat16

    Ld = _densify_coo(adj_row, adj_col, adj_val).astype(bf16)

    Psrc = jnp.transpose(
        jnp.squeeze(source, -1).astype(bf16).reshape(B, NPATCH, PATCH, N),
        (1, 3, 0, 2)).reshape(NPATCH, N, B * PATCH)

    bdve = _bd(W_ve).astype(bf16)
    bd0r_i = _bd(W0[:GRU, :GRU]).astype(bf16)
    bd0r_s = _bd(W0[GRU:, :GRU]).astype(bf16)
    bd0u_i = _bd(W0[:GRU, GRU:]).astype(bf16)
    bd0u_s = _bd(W0[GRU:, GRU:]).astype(bf16)
    bd1_i = _bd(W1[:GRU, :]).astype(bf16)
    bd1_s = _bd(W1[GRU:, :]).astype(bf16)
    W_out_pad = jnp.concatenate(
        [W_out, jnp.zeros((GRU - IB, OUTW), f32)], axis=0)
    bdout = _bd(W_out_pad).astype(bf16)

    petile = jnp.tile(pe[0, :NPATCH], (1, B)).reshape(NPATCH, 1, B * GRU)

    full = lambda shape: pl.BlockSpec(shape, lambda t: tuple(0 for _ in shape))
    out = pl.pallas_call(
        _step_body,
        grid=(NPATCH,),
        in_specs=[
            pl.BlockSpec((1, N, B * PATCH), lambda t: (t, 0, 0)),
            full((N, N)),
            full((B * PATCH, B * GRU)),
            full((B * GRU, B * GRU)),
            full((B * GRU, B * GRU)),
            full((B * GRU, B * GRU)),
            full((B * GRU, B * GRU)),
            full((B * GRU, B * GRU)),
            full((B * GRU, B * GRU)),
            full((B * GRU, B * OUTW)),
            pl.BlockSpec((1, 1, B * GRU), lambda t: (t, 0, 0)),
        ],
        out_specs=pl.BlockSpec((N, B * OUTW), lambda t: (0, 0)),
        out_shape=jax.ShapeDtypeStruct((N, B * OUTW), f32),
        scratch_shapes=[pltpu.VMEM((N, B * GRU), f32)],
        compiler_params=pltpu.CompilerParams(
            dimension_semantics=("arbitrary",)),
    )(Psrc, Ld, bdve, bd0r_i, bd0r_s, bd0u_i, bd0u_s, bd1_i, bd1_s, bdout,
      petile)

    return jnp.transpose(out.reshape(N, B, OUTW), (1, 2, 0))[..., None]
